# baseline (speedup 1.0000x reference)
import functools

import jax
import jax.numpy as jnp
from jax import lax
from jax.experimental import pallas as pl
from jax.experimental.pallas import tpu as pltpu
from jax.experimental.pallas import tpu_sc as plsc

V = 100000
END = 3
D = 64
B = 1024
S = 200
MAX_OOV = 256
NLANE = 16
NCHUNK = 13
SPAD = NCHUNK * NLANE
NFULL = S // NLANE
TAIL = S - NFULL * NLANE
NC, NS = 2, 16
NW = NC * NS
RPW = B // NW
WTOK = RPW * S


def _detok_body(seqs_hbm, oovs_hbm, vocab_hbm, str_out, lm_out,
                seqsall_v, lmall_v, oidx_v, vidx2_v, rows2_v, orows2_v,
                str_v, zidx_v, r0_v, o0_v, sem_g0, sem_g1, sem_o):
  wid = lax.axis_index("s") * NC + lax.axis_index("c")

  zidx_v[...] = jnp.zeros((NLANE,), jnp.int32)
  pltpu.async_copy(vocab_hbm.at[zidx_v], r0_v, sem_o).wait()
  pltpu.async_copy(oovs_hbm.at[zidx_v], o0_v, sem_o).wait()
  r0 = [r0_v[0, pl.ds(j * NLANE, NLANE)] for j in range(4)]
  o0 = [o0_v[0, pl.ds(j * NLANE, NLANE)] for j in range(4)]
  lane = lax.iota(jnp.int32, NLANE)
  tail_ok = lane < TAIL

  pltpu.sync_copy(seqs_hbm.at[pl.ds(wid * WTOK, WTOK)],
                  seqsall_v.at[pl.ds(0, WTOK)])

  zero_state = (jnp.int32(0), jnp.int32(0), jnp.int32(0))

  def stage_b(i, half):
    b = wid * RPW + i
    vidx_v = vidx2_v.at[half]
    orows_v = orows2_v.at[half]

    alive = jnp.ones((NLANE,), jnp.bool_)
    nflag = jnp.int32(0)
    cnt_vec = jnp.zeros((NLANE,), jnp.int32)
    ocnt_vec = jnp.zeros((NLANE,), jnp.int32)
    for c in range(NCHUNK):
      ids = seqsall_v[pl.ds(i * S + c * NLANE, NLANE)]
      valid = tail_ok if c == NCHUNK - 1 else None
      hits = ids == END
      if valid is not None:
        hits = jnp.logical_and(hits, valid)
      first = plsc.all_reduce_ffs(hits)
      lmb = jnp.logical_and(alive, lane <= first)
      lmall_v[pl.ds(i * S + c * NLANE, NLANE)] = jnp.where(
          lmb, 1.0, 0.0).astype(jnp.float32)
      alive = jnp.logical_and(alive, first > NLANE - 1)
      keep = jnp.logical_and(lmb, ids > 5)
      if valid is not None:
        keep = jnp.logical_and(keep, valid)
      is_oov = ids > V
      vkeep = jnp.logical_and(keep, jnp.logical_not(is_oov))
      vidx_v[pl.ds(c * NLANE, NLANE)] = jnp.where(vkeep, ids, 0)
      cnt_vec = cnt_vec + jnp.where(vkeep, 1, 0)
      okeep = jnp.logical_and(keep, is_oov)
      ocnt_vec = ocnt_vec + jnp.where(okeep, 1, 0)

      def oov_fn(nf, okeep=okeep, ids=ids, b=b):
        oidx_v[pl.ds(0, NLANE)] = jnp.where(
            okeep, b * MAX_OOV + (ids - V), 0)
        pltpu.async_copy(
            oovs_hbm.at[oidx_v.at[pl.ds(0, NLANE)]],
            orows_v.at[pl.ds(nf * NLANE, NLANE)], sem_o).wait()
        return nf + 1

      nflag = lax.cond(jnp.any(okeep), oov_fn, lambda nf: nf, nflag)

    return (jnp.sum(cnt_vec), nflag, jnp.sum(ocnt_vec))

  def stage_c(half, sem):
    vidx_v = vidx2_v.at[half]
    rows_v = rows2_v.at[half]
    descs = []
    for lo, n in ((0, 128), (128, S - 128)):
      descs.append(pltpu.async_copy(
          vocab_hbm.at[vidx_v.at[pl.ds(lo, n)]],
          rows_v.at[pl.ds(lo, n)], sem))
    return tuple(descs)

  def stage_d(i, half, descs, state):
    nvk, nflag, no_tot = state
    rows_v = rows2_v.at[half]
    orows_v = orows2_v.at[half]
    for d in descs:
      d.wait()

    acc = tuple(jnp.zeros((NLANE,), jnp.float32) for _ in range(4))

    def vacc_body(g, a):
      a = list(a)
      for r in range(NLANE):
        for j in range(4):
          a[j] = a[j] + rows_v[g * NLANE + r, pl.ds(j * NLANE, NLANE)]
      return tuple(a)
    acc = lax.fori_loop(0, NFULL, vacc_body, acc)
    acc = list(acc)
    for r in range(TAIL):
      for j in range(4):
        acc[j] = acc[j] + rows_v[NFULL * NLANE + r,
                                 pl.ds(j * NLANE, NLANE)]
    acc = tuple(acc)

    def oacc_body(r, a):
      return tuple(a[j] + orows_v[r, pl.ds(j * NLANE, NLANE)]
                   for j in range(4))
    acc = lax.fori_loop(0, nflag * NLANE, oacc_body, acc)

    vcorr = (S - nvk).astype(jnp.float32)
    ocf = (nflag * NLANE - no_tot).astype(jnp.float32)
    for j in range(4):
      out_j = acc[j] - vcorr * r0[j] - ocf * o0[j]
      str_v[pl.ds(i * D + j * NLANE, NLANE)] = out_j

  state0 = stage_b(0, 0)

  def pair_body(k, state_even):
    i = 2 * k
    descs0 = stage_c(0, sem_g0)
    state1 = stage_b(i + 1, 1)
    descs1 = stage_c(1, sem_g1)
    stage_d(i, 0, descs0, state_even)
    state_next = lax.cond(
        k < RPW // 2 - 1,
        lambda: stage_b(i + 2, 0),
        lambda: zero_state)
    stage_d(i + 1, 1, descs1, state1)
    return state_next

  lax.fori_loop(0, RPW // 2, pair_body, state0)
  pltpu.sync_copy(str_v, str_out.at[pl.ds(wid * RPW * D, RPW * D)])
  pltpu.sync_copy(lmall_v.at[pl.ds(0, WTOK)],
                  lm_out.at[pl.ds(wid * WTOK, WTOK)])


_detok = functools.partial(
    pl.kernel,
    out_type=(jax.ShapeDtypeStruct((B * D,), jnp.float32),
              jax.ShapeDtypeStruct((B * S,), jnp.float32)),
    mesh=plsc.VectorSubcoreMesh(
        core_axis_name="c", subcore_axis_name="s",
        num_cores=NC, num_subcores=NS),
    compiler_params=pltpu.CompilerParams(
        needs_layout_passes=False, use_tc_tiling_on_sc=False),
    scratch_types=[
        pltpu.VMEM((WTOK + NLANE,), jnp.int32),
        pltpu.VMEM((WTOK + NLANE,), jnp.float32),
        pltpu.VMEM((NLANE,), jnp.int32),
        pltpu.VMEM((2, SPAD), jnp.int32),
        pltpu.VMEM((2, SPAD, D), jnp.float32),
        pltpu.VMEM((2, SPAD, D), jnp.float32),
        pltpu.VMEM((RPW * D,), jnp.float32),
        pltpu.VMEM((NLANE,), jnp.int32),
        pltpu.VMEM((NLANE, D), jnp.float32),
        pltpu.VMEM((NLANE, D), jnp.float32),
        pltpu.SemaphoreType.DMA,
        pltpu.SemaphoreType.DMA,
        pltpu.SemaphoreType.DMA,
    ])(_detok_body)


@jax.jit
def kernel(input_seqs, oovs, vocab_table):
  strings_flat, lm_flat = _detok(
      input_seqs.reshape(-1), oovs.reshape(B * MAX_OOV, D), vocab_table)
  return strings_flat.reshape(B, D), lm_flat.reshape(B, S)
